# Initial kernel scaffold; baseline (speedup 1.0000x reference)
import jax
import jax.numpy as jnp
from jax.experimental import pallas as pl


def kernel(times, cat, mapping):
    raise NotImplementedError("write your pallas kernel here")


# speedup vs baseline: 91.4712x; 91.4712x over previous
import functools

import jax
import jax.numpy as jnp
from jax import lax
from jax.experimental import pallas as pl
from jax.experimental.pallas import tpu as pltpu
from jax.experimental.pallas import tpu_sc as plsc

NC = 2
NS = 16
NW = NC * NS
L = 16


def _make_kernel(B, T, V):
    total = B * T
    per_w = total // NW
    n_pass = 4
    chunk = per_w // n_pass
    n_vec = chunk // L

    mesh = plsc.VectorSubcoreMesh(core_axis_name="c", subcore_axis_name="s")

    @functools.partial(
        pl.kernel,
        out_type=(
            jax.ShapeDtypeStruct((total,), jnp.float32),
            jax.ShapeDtypeStruct((total,), jnp.int32),
        ),
        mesh=mesh,
        compiler_params=pltpu.CompilerParams(needs_layout_passes=False),
        scratch_types=[
            pltpu.VMEM((V,), jnp.int32),
            pltpu.VMEM((chunk,), jnp.float32),
            pltpu.VMEM((chunk,), jnp.float32),
            pltpu.VMEM((chunk,), jnp.int32),
            pltpu.VMEM((chunk,), jnp.int32),
        ],
    )
    def k(times_hbm, cat_hbm, map_hbm, dt_hbm, mp_hbm,
          map_v, times_v, dt_v, cat_v, mp_v):
        wid = lax.axis_index("s") * NC + lax.axis_index("c")
        pltpu.sync_copy(map_hbm, map_v)
        lanes = lax.iota(jnp.int32, L)
        for pi in range(n_pass):
            base = wid * per_w + pi * chunk
            pltpu.sync_copy(times_hbm.at[pl.ds(base, chunk)], times_v)
            pltpu.sync_copy(cat_hbm.at[pl.ds(base, chunk)], cat_v)

            def body(j, carry, lanes=lanes):
                sl = pl.ds(j * L, L)
                p = j * L + lanes
                t = lax.rem(p, T)
                prev_idx = jnp.where(t == 0, p, p - 1)
                cur = times_v[sl]
                prev = plsc.load_gather(times_v, [prev_idx])
                dt_v[sl] = cur - prev
                mp_v[sl] = plsc.load_gather(map_v, [cat_v[sl]])
                return carry

            lax.fori_loop(0, n_vec, body, 0)
            pltpu.sync_copy(dt_v, dt_hbm.at[pl.ds(base, chunk)])
            pltpu.sync_copy(mp_v, mp_hbm.at[pl.ds(base, chunk)])

    return k


def kernel(times, cat, mapping):
    B, T = times.shape
    V = mapping.shape[0]
    k = _make_kernel(B, T, V)
    deltas, mapped = k(times.reshape(-1), cat.reshape(-1), mapping)
    return deltas.reshape(B, T), mapped.reshape(B, T)


# speedup vs baseline: 121.0164x; 1.3230x over previous
import functools

import jax
import jax.numpy as jnp
from jax import lax
from jax.experimental import pallas as pl
from jax.experimental.pallas import tpu as pltpu
from jax.experimental.pallas import tpu_sc as plsc

NC = 2
NS = 16
NW = NC * NS
L = 16


def _make_kernel(B, T, V):
    total = B * T
    per_w = total // NW
    n_pass = 8
    chunk = per_w // n_pass
    n_vec = chunk // L

    mesh = plsc.VectorSubcoreMesh(core_axis_name="c", subcore_axis_name="s")

    @functools.partial(
        pl.kernel,
        out_type=(
            jax.ShapeDtypeStruct((total,), jnp.float32),
            jax.ShapeDtypeStruct((total,), jnp.int32),
        ),
        mesh=mesh,
        compiler_params=pltpu.CompilerParams(needs_layout_passes=False),
        scratch_types=[
            pltpu.VMEM((V,), jnp.int32),
            pltpu.VMEM((chunk,), jnp.float32),
            pltpu.VMEM((chunk,), jnp.float32),
            pltpu.VMEM((chunk,), jnp.int32),
            pltpu.VMEM((chunk,), jnp.int32),
            pltpu.VMEM((chunk,), jnp.float32),
            pltpu.VMEM((chunk,), jnp.float32),
            pltpu.VMEM((chunk,), jnp.int32),
            pltpu.VMEM((chunk,), jnp.int32),
            pltpu.SemaphoreType.DMA,
            pltpu.SemaphoreType.DMA,
            pltpu.SemaphoreType.DMA,
            pltpu.SemaphoreType.DMA,
            pltpu.SemaphoreType.DMA,
        ],
    )
    def k(times_hbm, cat_hbm, map_hbm, dt_hbm, mp_hbm,
          map_v, times_a, times_b, cat_a, cat_b, dt_a, dt_b, mp_a, mp_b,
          sem_t, sem_in_a, sem_in_b, sem_out_a, sem_out_b):
        wid = lax.axis_index("s") * NC + lax.axis_index("c")
        w_base = wid * per_w
        times_v = [times_a, times_b]
        cat_v = [cat_a, cat_b]
        dt_v = [dt_a, dt_b]
        mp_v = [mp_a, mp_b]
        sem_in = [sem_in_a, sem_in_b]
        sem_out = [sem_out_a, sem_out_b]

        t_desc = pltpu.async_copy(map_hbm, map_v, sem_t)

        def start_in(pi):
            base = w_base + pi * chunk
            s = pi % 2
            return (
                pltpu.async_copy(times_hbm.at[pl.ds(base, chunk)],
                                 times_v[s], sem_in[s]),
                pltpu.async_copy(cat_hbm.at[pl.ds(base, chunk)],
                                 cat_v[s], sem_in[s]),
            )

        in_descs = [None, None]
        out_descs = [None, None]
        in_descs[0] = start_in(0)
        lanes = lax.iota(jnp.int32, L)

        for pi in range(n_pass):
            s = pi % 2
            if pi + 1 < n_pass:
                in_descs[1 - s] = start_in(pi + 1)
            for d in in_descs[s]:
                d.wait()
            if pi == 0:
                t_desc.wait()
            if out_descs[s] is not None:
                for d in out_descs[s]:
                    d.wait()
            tv, cv, dv, mv = times_v[s], cat_v[s], dt_v[s], mp_v[s]

            @plsc.parallel_loop(0, n_vec, 1, unroll=8)
            def body(j, tv=tv, cv=cv, dv=dv, mv=mv):
                sl = pl.ds(j * L, L)
                p = j * L + lanes
                t = lax.rem(p, T)
                prev_idx = jnp.where(t == 0, p, p - 1)
                cur = tv[sl]
                prev = plsc.load_gather(tv, [prev_idx])
                dv[sl] = cur - prev
                mv[sl] = plsc.load_gather(map_v, [cv[sl]])

            base = w_base + pi * chunk
            out_descs[s] = (
                pltpu.async_copy(dv, dt_hbm.at[pl.ds(base, chunk)],
                                 sem_out[s]),
                pltpu.async_copy(mv, mp_hbm.at[pl.ds(base, chunk)],
                                 sem_out[s]),
            )

        for s in range(2):
            for d in out_descs[s]:
                d.wait()

    return k


def kernel(times, cat, mapping):
    B, T = times.shape
    V = mapping.shape[0]
    k = _make_kernel(B, T, V)
    deltas, mapped = k(times.reshape(-1), cat.reshape(-1), mapping)
    return deltas.reshape(B, T), mapped.reshape(B, T)


# speedup vs baseline: 150.3174x; 1.2421x over previous
import functools

import jax
import jax.numpy as jnp
from jax import lax
from jax.experimental import pallas as pl
from jax.experimental.pallas import tpu as pltpu
from jax.experimental.pallas import tpu_sc as plsc

NC = 2
NS = 16
NW = NC * NS
L = 16


def _make_kernel(B, T, V):
    rows_w = B // NW
    n_pass = 16
    rows_c = rows_w // n_pass
    chunk = rows_c * T
    n_vec = chunk // L

    mesh = plsc.VectorSubcoreMesh(core_axis_name="c", subcore_axis_name="s")

    @functools.partial(
        pl.kernel,
        out_type=(
            jax.ShapeDtypeStruct((B, T), jnp.float32),
            jax.ShapeDtypeStruct((B, T), jnp.int32),
        ),
        mesh=mesh,
        compiler_params=pltpu.CompilerParams(needs_layout_passes=False),
        scratch_types=[
            pltpu.VMEM((V,), jnp.int32),
            pltpu.VMEM((rows_c, T), jnp.float32),
            pltpu.VMEM((rows_c, T), jnp.float32),
            pltpu.VMEM((rows_c, T), jnp.int32),
            pltpu.VMEM((rows_c, T), jnp.int32),
            pltpu.VMEM((rows_c, T), jnp.float32),
            pltpu.VMEM((rows_c, T), jnp.float32),
            pltpu.VMEM((rows_c, T), jnp.int32),
            pltpu.VMEM((rows_c, T), jnp.int32),
            pltpu.SemaphoreType.DMA,
            pltpu.SemaphoreType.DMA,
            pltpu.SemaphoreType.DMA,
            pltpu.SemaphoreType.DMA,
            pltpu.SemaphoreType.DMA,
        ],
    )
    def k(times_hbm, cat_hbm, map_hbm, dt_hbm, mp_hbm,
          map_v, times_a, times_b, cat_a, cat_b, dt_a, dt_b, mp_a, mp_b,
          sem_t, sem_in_a, sem_in_b, sem_out_a, sem_out_b):
        wid = lax.axis_index("s") * NC + lax.axis_index("c")
        w_row = wid * rows_w
        times_v = [times_a, times_b]
        cat_v = [cat_a, cat_b]
        dt_v = [dt_a, dt_b]
        mp_v = [mp_a, mp_b]
        sem_in = [sem_in_a, sem_in_b]
        sem_out = [sem_out_a, sem_out_b]

        t_desc = pltpu.async_copy(map_hbm, map_v, sem_t)

        def start_in(pi):
            r0 = w_row + pi * rows_c
            s = pi % 2
            return (
                pltpu.async_copy(times_hbm.at[pl.ds(r0, rows_c), :],
                                 times_v[s], sem_in[s]),
                pltpu.async_copy(cat_hbm.at[pl.ds(r0, rows_c), :],
                                 cat_v[s], sem_in[s]),
            )

        in_descs = [None, None]
        out_descs = [None, None]
        in_descs[0] = start_in(0)
        lanes = lax.iota(jnp.int32, L)

        for pi in range(n_pass):
            s = pi % 2
            if pi + 1 < n_pass:
                in_descs[1 - s] = start_in(pi + 1)
            for d in in_descs[s]:
                d.wait()
            if pi == 0:
                t_desc.wait()
            if out_descs[s] is not None:
                for d in out_descs[s]:
                    d.wait()
            tv, cv, dv, mv = times_v[s], cat_v[s], dt_v[s], mp_v[s]

            @plsc.parallel_loop(0, n_vec, 1, unroll=8)
            def body(j, tv=tv, cv=cv, dv=dv, mv=mv):
                p = j * L + lanes
                r = lax.div(p, T)
                c = lax.rem(p, T)
                cp = jnp.where(c == 0, c, c - 1)
                cur = plsc.load_gather(tv, [r, c])
                prev = plsc.load_gather(tv, [r, cp])
                plsc.store_scatter(dv, [r, c], cur - prev)
                ci = plsc.load_gather(cv, [r, c])
                plsc.store_scatter(mv, [r, c], plsc.load_gather(map_v, [ci]))

            r0 = w_row + pi * rows_c
            out_descs[s] = (
                pltpu.async_copy(dv, dt_hbm.at[pl.ds(r0, rows_c), :],
                                 sem_out[s]),
                pltpu.async_copy(mv, mp_hbm.at[pl.ds(r0, rows_c), :],
                                 sem_out[s]),
            )

        for s in range(2):
            for d in out_descs[s]:
                d.wait()

    return k


def kernel(times, cat, mapping):
    B, T = times.shape
    V = mapping.shape[0]
    k = _make_kernel(B, T, V)
    return k(times, cat, mapping)
